# baseline (speedup 1.0000x reference)
import functools

import jax
import jax.numpy as jnp
from jax import lax
from jax.experimental import pallas as pl
from jax.experimental.pallas import tpu as pltpu
from jax.experimental.pallas import tpu_sc as plsc

_VOCAB = 1_000_000
_DIM = 64
_B = 4096
_L = 200

_NC = 2
_NS = 16
_NW = _NC * _NS
_RPW = _B // _NW
_NB = 4
_DW = 1
_DG = _NB - _DW


def _emb_body(idx_hbm, table_hbm, out_hbm, idx_v, rows_v, gsem, wsem):
    wid = lax.axis_index("s") * _NC + lax.axis_index("c")
    base = wid * _RPW

    pltpu.sync_copy(idx_hbm.at[pl.ds(base, _RPW)], idx_v)

    def start_gather(g, slot):
        pltpu.async_copy(table_hbm.at[idx_v.at[g]], rows_v.at[slot], gsem)

    def wait_gather(g, slot):
        pltpu.make_async_copy(
            table_hbm.at[idx_v.at[g]], rows_v.at[slot], gsem).wait()

    def start_wb(g, slot):
        pltpu.async_copy(rows_v.at[slot], out_hbm.at[base + g], wsem)

    def wait_wb(g, slot):
        pltpu.make_async_copy(
            rows_v.at[slot], out_hbm.at[base + g], wsem).wait()

    for g in range(_DG):
        start_gather(g, g)

    def step(g, b):
        wait_gather(g, b)
        start_wb(g, b)
        wait_wb(g - _DW, (b - _DW) % _NB)
        start_gather(g + _DG, (b + _DG) % _NB)

    for g in range(_NB):
        wait_gather(g, g)
        start_wb(g, g)
        if g >= _DW:
            wait_wb(g - _DW, (g - _DW) % _NB)
        start_gather(g + _DG, (g + _DG) % _NB)

    @pl.loop(_NB, _RPW - _NB, step=_NB)
    def _steady(go):
        for b in range(_NB):
            step(go + b, b)

    for g in range(_RPW - _NB, _RPW):
        wait_gather(g, g % _NB)
        start_wb(g, g % _NB)
        wait_wb(g - _DW, (g - _DW) % _NB)
        if g + _DG < _RPW:
            start_gather(g + _DG, (g + _DG) % _NB)

    for g in range(_RPW - _DW, _RPW):
        wait_wb(g, g % _NB)


@jax.jit
def _embedding_lookup(idx, table):
    mesh = plsc.VectorSubcoreMesh(core_axis_name="c", subcore_axis_name="s")
    fn = functools.partial(
        pl.kernel,
        out_type=jax.ShapeDtypeStruct((_B, _L, _DIM), jnp.float32),
        mesh=mesh,
        scratch_types=[
            pltpu.VMEM((_RPW, _L), jnp.int32),
            pltpu.VMEM((_NB, _L, _DIM), jnp.float32),
            pltpu.SemaphoreType.DMA,
            pltpu.SemaphoreType.DMA,
        ],
        compiler_params=pltpu.CompilerParams(use_tc_tiling_on_sc=False),
    )(_emb_body)
    return fn(idx, table)


def kernel(inputs, table):
    return _embedding_lookup(inputs.astype(jnp.int32), table)


# speedup vs baseline: 1.3335x; 1.3335x over previous
import functools

import jax
import jax.numpy as jnp
from jax import lax
from jax.experimental import pallas as pl
from jax.experimental.pallas import tpu as pltpu
from jax.experimental.pallas import tpu_sc as plsc

_VOCAB = 1_000_000
_DIM = 64
_PAD = 128
_B = 4096
_L = 200

_NC = 2
_NS = 16
_NW = _NC * _NS
_RPW = _B // _NW
_NB = 4
_DW = 1
_DG = _NB - _DW


def _emb_body(idx_hbm, table_hbm, out_hbm, idx_v, rows_v, gsem, wsem):
    wid = lax.axis_index("s") * _NC + lax.axis_index("c")
    base = wid * _RPW

    pltpu.sync_copy(idx_hbm.at[pl.ds(base, _RPW)], idx_v)

    def start_gather(g, slot):
        pltpu.async_copy(table_hbm.at[idx_v.at[g]], rows_v.at[slot], gsem)

    def wait_gather(g, slot):
        pltpu.make_async_copy(
            table_hbm.at[idx_v.at[g]], rows_v.at[slot], gsem).wait()

    def start_wb(g, slot):
        pltpu.async_copy(
            rows_v.at[slot], out_hbm.at[base + g, :, pl.ds(0, _DIM)], wsem)

    def wait_wb(g, slot):
        pltpu.make_async_copy(
            rows_v.at[slot], out_hbm.at[base + g, :, pl.ds(0, _DIM)],
            wsem).wait()

    for g in range(_DG):
        start_gather(g, g)

    def step(g, b):
        wait_gather(g, b)
        start_wb(g, b)
        wait_wb(g - _DW, (b - _DW) % _NB)
        start_gather(g + _DG, (b + _DG) % _NB)

    for g in range(_NB):
        wait_gather(g, g)
        start_wb(g, g)
        if g >= _DW:
            wait_wb(g - _DW, (g - _DW) % _NB)
        start_gather(g + _DG, (g + _DG) % _NB)

    @pl.loop(_NB, _RPW - _NB, step=_NB)
    def _steady(go):
        for b in range(_NB):
            step(go + b, b)

    for g in range(_RPW - _NB, _RPW):
        wait_gather(g, g % _NB)
        start_wb(g, g % _NB)
        wait_wb(g - _DW, (g - _DW) % _NB)
        if g + _DG < _RPW:
            start_gather(g + _DG, (g + _DG) % _NB)

    for g in range(_RPW - _DW, _RPW):
        wait_wb(g, g % _NB)


@jax.jit
def _embedding_lookup(idx, table):
    mesh = plsc.VectorSubcoreMesh(core_axis_name="c", subcore_axis_name="s")
    fn = functools.partial(
        pl.kernel,
        out_type=jax.ShapeDtypeStruct((_B, _L, _PAD), jnp.float32),
        mesh=mesh,
        scratch_types=[
            pltpu.VMEM((_RPW, _L), jnp.int32),
            pltpu.VMEM((_NB, _L, _DIM), jnp.float32),
            pltpu.SemaphoreType.DMA,
            pltpu.SemaphoreType.DMA,
        ],
        compiler_params=pltpu.CompilerParams(use_tc_tiling_on_sc=False),
    )(_emb_body)
    return fn(idx, table)


def kernel(inputs, table):
    out128 = _embedding_lookup(inputs.astype(jnp.int32), table)
    return lax.slice(out128, (0, 0, 0), (_B, _L, _DIM))
